# baseline (speedup 1.0000x reference)
import functools

import numpy as _np

import jax
import jax.numpy as jnp
from jax import lax
from jax.experimental import pallas as pl
from jax.experimental.pallas import tpu as pltpu
from jax.experimental.pallas import tpu_sc as plsc

_NC2 = 1000
_NC3 = 4000
_ND = 12000
_H = 128
_B = 64
_S = 50
_T = 4

_SENT = 1 << 20

_H2CAP = 128
_H3CAP = 256

_SC_CORES = 2
_SC_SUBCORES = 16
_GPW = 128
_NW_USED = (_B * _S) // _GPW


def _gamma_sc_body(dw_hbm, idx_hbm, out_hbm, idx_v, val_v, sem):
    wid = lax.axis_index("s") * _SC_CORES + lax.axis_index("c")

    @pl.when(wid < _NW_USED)
    def _():
        base = wid * _GPW
        pltpu.sync_copy(idx_hbm.at[pl.ds(base, _GPW)], idx_v)
        pltpu.async_copy(dw_hbm.at[idx_v], val_v, sem).wait()
        pltpu.sync_copy(val_v, out_hbm.at[pl.ds(base, _GPW)])


def _gamma_gather(dw_flat, idx_flat):
    return pl.kernel(
        _gamma_sc_body,
        out_type=jax.ShapeDtypeStruct((_B * _S,), jnp.float32),
        mesh=plsc.VectorSubcoreMesh(core_axis_name="c", subcore_axis_name="s"),
        scratch_types=[
            pltpu.VMEM((_GPW,), jnp.int32),
            pltpu.VMEM((_GPW,), jnp.float32),
            pltpu.SemaphoreType.DMA,
        ],
    )(dw_flat, idx_flat)


def _tc_body(
    gamma_ref, ints_ref, gamma_full_ref, ints_full_ref,
    v_c2_ref, v_c3_ref, v_d_ref, Rw_ref,
    WihT_ref, WhhT_ref, bih_ref, bhh_ref,
    W1aT_ref, b1a_ref, W1b_ref, b1b_ref,
    W2aT_ref, b2a_ref, W2b_ref, b2b_ref,
    W3aT_ref, b3a_ref, W3b_ref, b3b_ref,
    alpha_ref, h_ref, C2_ref, C3_ref,
    h_st, C2_st, C3_st, i2h, v2h, i3h, v3h, gi_all,
):
    s = pl.program_id(0)

    f32 = jnp.float32
    bf16 = jnp.bfloat16
    dot = lambda x, w: jnp.dot(x.astype(bf16), w, preferred_element_type=f32)
    q = lambda x: x.astype(bf16).astype(f32)

    @pl.when(s == 0)
    def _init():
        h_st[...] = jnp.zeros_like(h_st)
        C2_st[...] = jnp.zeros_like(C2_st)
        C3_st[...] = jnp.zeros_like(C3_st)
        i2h[...] = jnp.full_like(i2h, -1)
        i3h[...] = jnp.full_like(i3h, -1)
        v2h[...] = jnp.zeros_like(v2h)
        v3h[...] = jnp.zeros_like(v3h)
        g_all = gamma_full_ref[...]
        r_all = ints_full_ref[:, :, 5:6]
        vd_all = g_all * v_d_ref[...]
        vr_all = jnp.where(r_all == 1, Rw_ref[1:2, :], Rw_ref[0:1, :])
        x_all = jnp.concatenate([vd_all, vr_all], axis=2).reshape(
            _S * _B, 2 * _H)
        gi_all[...] = dot(x_all, WihT_ref[...])

    gamma = gamma_ref[0]
    ints = ints_ref[0]
    vd_t = gamma * v_d_ref[...]
    r = ints[:, 5:6]
    vr_t = jnp.where(r == 1, Rw_ref[1:2, :], Rw_ref[0:1, :])

    h = h_st[...]
    gi = gi_all[pl.ds(s * _B, _B), :] + bih_ref[...]
    gh = dot(h, WhhT_ref[...]) + bhh_ref[...]
    r_g = jax.nn.sigmoid(gi[:, :_H] + gh[:, :_H])
    z_g = jax.nn.sigmoid(gi[:, _H:2 * _H] + gh[:, _H:2 * _H])
    n_g = jnp.tanh(gi[:, 2 * _H:] + r_g * gh[:, 2 * _H:])
    h_new = (1.0 - z_g) * n_g + z_g * h
    h_st[...] = h_new
    h_ref[0] = h_new

    a = jnp.maximum(dot(h_new, W1aT_ref[...]) + b1a_ref[...], 0.0)
    alpha = (jnp.sum(q(a) * W1b_ref[...].astype(f32), axis=1, keepdims=True)
             + b1b_ref[...])
    lane = lax.broadcasted_iota(jnp.int32, (_B, _S), 1)
    alpha_ref[...] = jnp.where(lane == s, alpha, alpha_ref[...])

    c2 = ints[:, 0:1]
    c3_all = ints[:, 1:5]

    li2 = lax.broadcasted_iota(jnp.int32, (_B, _H2CAP), 1)
    m2h = i2h[...] == c2
    beta2 = jnp.sum(jnp.where(m2h, v2h[...], 0.0),
                    axis=1, keepdims=True)

    li3 = lax.broadcasted_iota(jnp.int32, (_B, _H3CAP), 1)
    i3cur = i3h[...]
    v3cur = v3h[...]
    c3t = []
    m3h = []
    beta3 = []
    masks = []
    for t in range(_T):
        ct = c3_all[:, t:t + 1]
        c3t.append(ct)
        m = i3cur == ct
        m3h.append(m)
        beta3.append(jnp.sum(jnp.where(m, v3cur, 0.0),
                             axis=1, keepdims=True))
        masks.append(jnp.where(ct != _SENT, 1.0, 0.0))
    denom = jnp.maximum(masks[0] + masks[1] + masks[2] + masks[3], 1e-6)
    beta3_bar = sum(beta3[t] * (masks[t] / denom) for t in range(_T))

    v_c2_t = beta2 * v_c2_ref[...]
    v_c3_bar = beta3_bar * v_c3_ref[...]
    feat2 = jnp.concatenate([v_c2_t, v_c3_bar, vd_t, vr_t], axis=1)
    a2 = jnp.maximum(dot(feat2, W2aT_ref[...]) + b2a_ref[...], 0.0)
    new_c2 = (jnp.sum(q(a2) * W2b_ref[...].astype(f32), axis=1, keepdims=True)
              + b2b_ref[...])

    feat3 = jnp.concatenate(
        [jnp.concatenate(
            [v_c2_t, beta3[t] * v_c3_ref[...], vd_t, vr_t], axis=1)
         for t in range(_T)], axis=0)
    a3 = jnp.maximum(dot(feat3, W3aT_ref[...]) + b3a_ref[...], 0.0)
    new_c3 = (jnp.sum(q(a3) * W3b_ref[...].astype(f32), axis=1, keepdims=True)
              + b3b_ref[...])

    stored3 = []
    for t in range(_T):
        kt = sum(jnp.where(c3t[t] == c3t[u], 1.0, 0.0) for u in range(_T))
        ssum = sum(jnp.where(c3t[t] == c3t[u], new_c3[u * _B:(u + 1) * _B], 0.0)
                   for u in range(_T))
        stored3.append(ssum - (kt - 1.0) * beta3[t])

    base = _T * s
    i3n, v3n = i3cur, v3cur
    for t in range(_T):
        i3n = jnp.where(m3h[t], -3, i3n)
    for t in range(_T):
        keep = c3t[t] != _SENT
        for u in range(t + 1, _T):
            keep = keep & (c3t[t] != c3t[u])
        at = li3 == base + t
        i3n = jnp.where(at, jnp.where(keep, c3t[t], -2), i3n)
        v3n = jnp.where(at, stored3[t], v3n)
    i3h[...] = i3n
    v3h[...] = v3n
    i2h[...] = jnp.where(m2h, -3, i2h[...])
    i2h[...] = jnp.where(li2 == s, c2, i2h[...])
    v2h[...] = jnp.where(li2 == s, new_c2, v2h[...])

    iota2 = lax.broadcasted_iota(jnp.int32, (_B, _NC2), 1)
    C2n = jnp.where(iota2 == c2, new_c2, C2_st[...])
    C2_st[...] = C2n
    C2_ref[0] = C2n

    iota3 = lax.broadcasted_iota(jnp.int32, (_B, _NC3 + 1), 1)
    C3n = C3_st[...]
    for t in range(_T):
        C3n = jnp.where(iota3 == c3t[t], stored3[t], C3n)
    C3_st[...] = C3n
    C3_ref[0] = C3n


def _run_tc(gamma_sib, ints_sb,
            v_c2, v_c3, v_d, R_w,
            WihT, WhhT, b_ih, b_hh,
            W1aT, b1a, W1b, b1b,
            W2aT, b2a, W2b, b2b,
            W3aT, b3a, W3b, b3b,
            interpret=False):
    full = lambda shape: pl.BlockSpec(shape, lambda s: (0,) * len(shape))
    step3 = lambda shape: pl.BlockSpec(shape, lambda s: (s, 0, 0))
    alpha, h_sb, c2_sb, c3_sb = pl.pallas_call(
        _tc_body,
        grid=(_S,),
        in_specs=[
            step3((1, _B, 1)), step3((1, _B, _T + 2)),
            full((_S, _B, 1)), full((_S, _B, _T + 2)),
            full((1, _H)), full((1, _H)), full((1, _H)), full((2, _H)),
            full((2 * _H, 3 * _H)), full((_H, 3 * _H)),
            full((1, 3 * _H)), full((1, 3 * _H)),
            full((_H, _H)), full((1, _H)), full((1, _H)), full((1, 1)),
            full((4 * _H, _H)), full((1, _H)), full((1, _H)), full((1, 1)),
            full((4 * _H, _H)), full((1, _H)), full((1, _H)), full((1, 1)),
        ],
        out_specs=[
            pl.BlockSpec((_B, _S), lambda s: (0, 0)),
            step3((1, _B, _H)),
            step3((1, _B, _NC2)),
            step3((1, _B, _NC3 + 1)),
        ],
        out_shape=[
            jax.ShapeDtypeStruct((_B, _S), jnp.float32),
            jax.ShapeDtypeStruct((_S, _B, _H), jnp.float32),
            jax.ShapeDtypeStruct((_S, _B, _NC2), jnp.float32),
            jax.ShapeDtypeStruct((_S, _B, _NC3 + 1), jnp.float32),
        ],
        scratch_shapes=[
            pltpu.VMEM((_B, _H), jnp.float32),
            pltpu.VMEM((_B, _NC2), jnp.float32),
            pltpu.VMEM((_B, _NC3 + 1), jnp.float32),
            pltpu.VMEM((_B, _H2CAP), jnp.int32),
            pltpu.VMEM((_B, _H2CAP), jnp.float32),
            pltpu.VMEM((_B, _H3CAP), jnp.int32),
            pltpu.VMEM((_B, _H3CAP), jnp.float32),
            pltpu.VMEM((_S * _B, 3 * _H), jnp.float32),
        ],
        interpret=interpret,
    )(gamma_sib, ints_sb, gamma_sib, ints_sb,
      v_c2, v_c3, v_d, R_w,
      WihT, WhhT, b_ih, b_hh,
      W1aT, b1a, W1b, b1b,
      W2aT, b2a, W2b, b2b,
      W3aT, b3a, W3b, b3b)
    return (alpha, jnp.swapaxes(h_sb, 0, 1), jnp.swapaxes(c2_sb, 0, 1),
            jnp.swapaxes(c3_sb, 0, 1))


def kernel(v_c2, v_c3, v_d, D_w, R_w, W_ih, W_hh, b_ih, b_hh, W1a, b1a, W1b, b1b, W2a, b2a, W2b, b2b, W3a, b3a, W3b, b3b, c2_seq, c3_seq, d_seq, r_seq):
    f32 = jnp.float32
    i32 = jnp.int32
    gam = _gamma_gather(D_w.reshape(-1).astype(f32),
                        d_seq.astype(i32).reshape(-1))
    gamma_sib = gam.reshape(_B, _S).T.reshape(_S, _B, 1)

    c3i = c3_seq.astype(i32)
    ints_sb = jnp.concatenate([
        c2_seq.astype(i32).T.reshape(_S, _B, 1),
        jnp.transpose(jnp.where(c3i == 0, _SENT, c3i), (1, 0, 2)),
        r_seq.astype(i32).T.reshape(_S, _B, 1),
    ], axis=-1)

    bf16 = jnp.bfloat16
    alpha, h_seq, C2_seq, C3_seq = _run_tc(
        gamma_sib, ints_sb,
        v_c2.reshape(1, _H), v_c3.reshape(1, _H), v_d.reshape(1, _H),
        R_w.astype(f32),
        W_ih.T.astype(bf16), W_hh.T.astype(bf16),
        b_ih.reshape(1, -1), b_hh.reshape(1, -1),
        W1a.T.astype(bf16), b1a.reshape(1, -1),
        W1b.reshape(1, _H).astype(bf16), b1b.reshape(1, 1),
        W2a.T.astype(bf16), b2a.reshape(1, -1),
        W2b.reshape(1, _H).astype(bf16), b2b.reshape(1, 1),
        W3a.T.astype(bf16), b3a.reshape(1, -1),
        W3b.reshape(1, _H).astype(bf16), b3b.reshape(1, 1),
    )
    return alpha, h_seq, C2_seq, C3_seq


# speedup vs baseline: 1.0137x; 1.0137x over previous
import functools

import numpy as _np

import jax
import jax.numpy as jnp
from jax import lax
from jax.experimental import pallas as pl
from jax.experimental.pallas import tpu as pltpu
from jax.experimental.pallas import tpu_sc as plsc

_NC2 = 1000
_NC3 = 4000
_ND = 12000
_H = 128
_B = 64
_S = 50
_T = 4

_SENT = 1 << 20

_H2CAP = 128
_H3CAP = 256

_SC_CORES = 2
_SC_SUBCORES = 16
_GPW = 128
_NW_USED = (_B * _S) // _GPW


def _gamma_sc_body(dw_hbm, idx_hbm, out_hbm, idx_v, val_v, sem):
    wid = lax.axis_index("s") * _SC_CORES + lax.axis_index("c")

    @pl.when(wid < _NW_USED)
    def _():
        base = wid * _GPW
        pltpu.sync_copy(idx_hbm.at[pl.ds(base, _GPW)], idx_v)
        pltpu.async_copy(dw_hbm.at[idx_v], val_v, sem).wait()
        pltpu.sync_copy(val_v, out_hbm.at[pl.ds(base, _GPW)])


def _gamma_gather(dw_flat, idx_flat):
    return pl.kernel(
        _gamma_sc_body,
        out_type=jax.ShapeDtypeStruct((_B * _S,), jnp.float32),
        mesh=plsc.VectorSubcoreMesh(core_axis_name="c", subcore_axis_name="s"),
        scratch_types=[
            pltpu.VMEM((_GPW,), jnp.int32),
            pltpu.VMEM((_GPW,), jnp.float32),
            pltpu.SemaphoreType.DMA,
        ],
    )(dw_flat, idx_flat)


def _tc_body(
    gamma_ref, ints_ref,
    v_c2_ref, v_c3_ref, v_d_ref, Rw_ref,
    WihT_ref, WhhT_ref, bih_ref, bhh_ref,
    W1aT_ref, b1a_ref, W1b_ref, b1b_ref,
    W2aT_ref, b2a_ref, W2b_ref, b2b_ref,
    W3aT_ref, b3a_ref, W3b_ref, b3b_ref,
    alpha_ref, h_ref, C2_ref, C3_ref,
    h_st, C2_st, C3_st, i2h, v2h, i3h, v3h,
):
    s = pl.program_id(0)

    f32 = jnp.float32
    bf16 = jnp.bfloat16
    dot = lambda x, w: jnp.dot(x.astype(bf16), w, preferred_element_type=f32)
    q = lambda x: x.astype(bf16).astype(f32)

    @pl.when(s == 0)
    def _init():
        h_st[...] = jnp.zeros_like(h_st)
        C2_st[...] = jnp.zeros_like(C2_st)
        C3_st[...] = jnp.zeros_like(C3_st)
        i2h[...] = jnp.full_like(i2h, -1)
        i3h[...] = jnp.full_like(i3h, -1)
        v2h[...] = jnp.zeros_like(v2h)
        v3h[...] = jnp.zeros_like(v3h)

    gamma = gamma_ref[0]
    ints = ints_ref[0]
    vd_t = gamma * v_d_ref[...]
    r = ints[:, 5:6]
    vr_t = jnp.where(r == 1, Rw_ref[1:2, :], Rw_ref[0:1, :])

    h = h_st[...]
    x = jnp.concatenate([vd_t, vr_t], axis=1)
    gi = dot(x, WihT_ref[...]) + bih_ref[...]
    gh = dot(h, WhhT_ref[...]) + bhh_ref[...]
    r_g = jax.nn.sigmoid(gi[:, :_H] + gh[:, :_H])
    z_g = jax.nn.sigmoid(gi[:, _H:2 * _H] + gh[:, _H:2 * _H])
    n_g = jnp.tanh(gi[:, 2 * _H:] + r_g * gh[:, 2 * _H:])
    h_new = (1.0 - z_g) * n_g + z_g * h
    h_st[...] = h_new
    h_ref[0] = h_new

    a = jnp.maximum(dot(h_new, W1aT_ref[...]) + b1a_ref[...], 0.0)
    alpha = (jnp.sum(q(a) * W1b_ref[...].astype(f32), axis=1, keepdims=True)
             + b1b_ref[...])
    lane = lax.broadcasted_iota(jnp.int32, (_B, _S), 1)
    alpha_ref[...] = jnp.where(lane == s, alpha, alpha_ref[...])

    c2 = ints[:, 0:1]
    c3_all = ints[:, 1:5]

    li2 = lax.broadcasted_iota(jnp.int32, (_B, _H2CAP), 1)
    m2h = i2h[...] == c2
    beta2 = jnp.sum(jnp.where(m2h, v2h[...], 0.0),
                    axis=1, keepdims=True)

    li3 = lax.broadcasted_iota(jnp.int32, (_B, _H3CAP), 1)
    i3cur = i3h[...]
    v3cur = v3h[...]
    c3t = []
    m3h = []
    beta3 = []
    masks = []
    for t in range(_T):
        ct = c3_all[:, t:t + 1]
        c3t.append(ct)
        m = i3cur == ct
        m3h.append(m)
        beta3.append(jnp.sum(jnp.where(m, v3cur, 0.0),
                             axis=1, keepdims=True))
        masks.append(jnp.where(ct != _SENT, 1.0, 0.0))
    denom = jnp.maximum(masks[0] + masks[1] + masks[2] + masks[3], 1e-6)
    beta3_bar = sum(beta3[t] * (masks[t] / denom) for t in range(_T))

    v_c2_t = beta2 * v_c2_ref[...]
    v_c3_bar = beta3_bar * v_c3_ref[...]
    feat2 = jnp.concatenate([v_c2_t, v_c3_bar, vd_t, vr_t], axis=1)
    a2 = jnp.maximum(dot(feat2, W2aT_ref[...]) + b2a_ref[...], 0.0)
    new_c2 = (jnp.sum(q(a2) * W2b_ref[...].astype(f32), axis=1, keepdims=True)
              + b2b_ref[...])

    feat3 = jnp.concatenate(
        [jnp.concatenate(
            [v_c2_t, beta3[t] * v_c3_ref[...], vd_t, vr_t], axis=1)
         for t in range(_T)], axis=0)
    a3 = jnp.maximum(dot(feat3, W3aT_ref[...]) + b3a_ref[...], 0.0)
    new_c3 = (jnp.sum(q(a3) * W3b_ref[...].astype(f32), axis=1, keepdims=True)
              + b3b_ref[...])

    stored3 = []
    for t in range(_T):
        kt = sum(jnp.where(c3t[t] == c3t[u], 1.0, 0.0) for u in range(_T))
        ssum = sum(jnp.where(c3t[t] == c3t[u], new_c3[u * _B:(u + 1) * _B], 0.0)
                   for u in range(_T))
        stored3.append(ssum - (kt - 1.0) * beta3[t])

    base = _T * s
    i3n, v3n = i3cur, v3cur
    for t in range(_T):
        i3n = jnp.where(m3h[t], -3, i3n)
    for t in range(_T):
        keep = c3t[t] != _SENT
        for u in range(t + 1, _T):
            keep = keep & (c3t[t] != c3t[u])
        at = li3 == base + t
        i3n = jnp.where(at, jnp.where(keep, c3t[t], -2), i3n)
        v3n = jnp.where(at, stored3[t], v3n)
    i3h[...] = i3n
    v3h[...] = v3n
    i2h[...] = jnp.where(m2h, -3, i2h[...])
    i2h[...] = jnp.where(li2 == s, c2, i2h[...])
    v2h[...] = jnp.where(li2 == s, new_c2, v2h[...])

    iota2 = lax.broadcasted_iota(jnp.int32, (_B, _NC2), 1)
    C2n = jnp.where(iota2 == c2, new_c2, C2_st[...])
    C2_st[...] = C2n
    C2_ref[0] = C2n

    iota3 = lax.broadcasted_iota(jnp.int32, (_B, _NC3 + 1), 1)
    C3n = C3_st[...]
    for t in range(_T):
        C3n = jnp.where(iota3 == c3t[t], stored3[t], C3n)
    C3_st[...] = C3n
    C3_ref[0] = C3n


def _run_tc(gamma_sib, ints_sb,
            v_c2, v_c3, v_d, R_w,
            WihT, WhhT, b_ih, b_hh,
            W1aT, b1a, W1b, b1b,
            W2aT, b2a, W2b, b2b,
            W3aT, b3a, W3b, b3b,
            interpret=False):
    full = lambda shape: pl.BlockSpec(shape, lambda s: (0,) * len(shape))
    step3 = lambda shape: pl.BlockSpec(shape, lambda s: (s, 0, 0))
    alpha, h_sb, c2_sb, c3_sb = pl.pallas_call(
        _tc_body,
        grid=(_S,),
        in_specs=[
            step3((1, _B, 1)), step3((1, _B, _T + 2)),
            full((1, _H)), full((1, _H)), full((1, _H)), full((2, _H)),
            full((2 * _H, 3 * _H)), full((_H, 3 * _H)),
            full((1, 3 * _H)), full((1, 3 * _H)),
            full((_H, _H)), full((1, _H)), full((1, _H)), full((1, 1)),
            full((4 * _H, _H)), full((1, _H)), full((1, _H)), full((1, 1)),
            full((4 * _H, _H)), full((1, _H)), full((1, _H)), full((1, 1)),
        ],
        out_specs=[
            pl.BlockSpec((_B, _S), lambda s: (0, 0)),
            step3((1, _B, _H)),
            step3((1, _B, _NC2)),
            step3((1, _B, _NC3 + 1)),
        ],
        out_shape=[
            jax.ShapeDtypeStruct((_B, _S), jnp.float32),
            jax.ShapeDtypeStruct((_S, _B, _H), jnp.float32),
            jax.ShapeDtypeStruct((_S, _B, _NC2), jnp.float32),
            jax.ShapeDtypeStruct((_S, _B, _NC3 + 1), jnp.float32),
        ],
        scratch_shapes=[
            pltpu.VMEM((_B, _H), jnp.float32),
            pltpu.VMEM((_B, _NC2), jnp.float32),
            pltpu.VMEM((_B, _NC3 + 1), jnp.float32),
            pltpu.VMEM((_B, _H2CAP), jnp.int32),
            pltpu.VMEM((_B, _H2CAP), jnp.float32),
            pltpu.VMEM((_B, _H3CAP), jnp.int32),
            pltpu.VMEM((_B, _H3CAP), jnp.float32),
        ],
        interpret=interpret,
    )(gamma_sib, ints_sb,
      v_c2, v_c3, v_d, R_w,
      WihT, WhhT, b_ih, b_hh,
      W1aT, b1a, W1b, b1b,
      W2aT, b2a, W2b, b2b,
      W3aT, b3a, W3b, b3b)
    return (alpha, jnp.swapaxes(h_sb, 0, 1), jnp.swapaxes(c2_sb, 0, 1),
            jnp.swapaxes(c3_sb, 0, 1))


def kernel(v_c2, v_c3, v_d, D_w, R_w, W_ih, W_hh, b_ih, b_hh, W1a, b1a, W1b, b1b, W2a, b2a, W2b, b2b, W3a, b3a, W3b, b3b, c2_seq, c3_seq, d_seq, r_seq):
    f32 = jnp.float32
    i32 = jnp.int32
    gam = _gamma_gather(D_w.reshape(-1).astype(f32),
                        d_seq.astype(i32).reshape(-1))
    gamma_sib = gam.reshape(_B, _S).T.reshape(_S, _B, 1)

    c3i = c3_seq.astype(i32)
    ints_sb = jnp.concatenate([
        c2_seq.astype(i32).T.reshape(_S, _B, 1),
        jnp.transpose(jnp.where(c3i == 0, _SENT, c3i), (1, 0, 2)),
        r_seq.astype(i32).T.reshape(_S, _B, 1),
    ], axis=-1)

    bf16 = jnp.bfloat16
    alpha, h_seq, C2_seq, C3_seq = _run_tc(
        gamma_sib, ints_sb,
        v_c2.reshape(1, _H), v_c3.reshape(1, _H), v_d.reshape(1, _H),
        R_w.astype(f32),
        W_ih.T.astype(bf16), W_hh.T.astype(bf16),
        b_ih.reshape(1, -1), b_hh.reshape(1, -1),
        W1a.T.astype(bf16), b1a.reshape(1, -1),
        W1b.reshape(1, _H).astype(bf16), b1b.reshape(1, 1),
        W2a.T.astype(bf16), b2a.reshape(1, -1),
        W2b.reshape(1, _H).astype(bf16), b2b.reshape(1, 1),
        W3a.T.astype(bf16), b3a.reshape(1, -1),
        W3b.reshape(1, _H).astype(bf16), b3b.reshape(1, 1),
    )
    return alpha, h_seq, C2_seq, C3_seq
